# baseline (speedup 1.0000x reference)
import functools

import jax
import jax.numpy as jnp
from jax import lax
from jax.experimental import pallas as pl
from jax.experimental.pallas import tpu as pltpu
from jax.experimental.pallas import tpu_sc as plsc

TOTAL = 32768
D = 256
NSEG = 16
LANES = 16
NVEC = D // LANES

TC_ROWS = 16384
SC_ROWS = TOTAL - TC_ROWS

NC = 2
NS = 16
NW = NC * NS
ROWS_PER_W = SC_ROWS // NW
CHUNK = 128
NCHUNK = ROWS_PER_W // CHUNK

TBLK = 4096
TC_STEPS = TC_ROWS // TBLK


def _sc_segsum_body(
    flat_hbm, cu_hbm, psum_hbm, cu_v, buf0, buf1, acc_v, sem0, sem1
):
  cid = lax.axis_index("c")
  sid = lax.axis_index("s")
  wid = sid * NC + cid
  base = TC_ROWS + wid * ROWS_PER_W

  pltpu.sync_copy(cu_hbm.at[pl.ds(0, LANES)], cu_v.at[pl.ds(0, LANES)])
  cu_v[pl.ds(LANES, LANES)] = jnp.full((LANES,), TOTAL, jnp.int32)

  def cu_at(k):
    return cu_v[pl.ds(k, LANES)][0]

  zeros = jnp.zeros((LANES,), jnp.float32)

  @pl.loop(0, NSEG)
  def _zero(s):
    for j in range(NVEC):
      acc_v[s, pl.ds(j * LANES, LANES)] = zeros

  bufs = (buf0, buf1)
  sems = (sem0, sem1)

  pltpu.async_copy(flat_hbm.at[pl.ds(base, CHUNK)], bufs[0], sems[0])

  def compute(buf, row0):
    @pl.loop(0, NSEG)
    def _seg(k):
      lo = jnp.maximum(cu_at(k), row0) - row0
      hi = jnp.minimum(cu_at(k + 1), row0 + CHUNK) - row0

      @pl.when(hi > lo)
      def _():
        init = tuple(acc_v[k, pl.ds(j * LANES, LANES)] for j in range(NVEC))

        @pl.loop(lo, hi, init_carry=init)
        def acc(r, carry):
          return tuple(
              carry[j] + buf[r, pl.ds(j * LANES, LANES)]
              for j in range(NVEC)
          )

        for j in range(NVEC):
          acc_v[k, pl.ds(j * LANES, LANES)] = acc[j]

  @pl.loop(0, NCHUNK, step=2)
  def _pair(ch):
    for b in range(2):
      cur = ch + b

      @pl.when(cur + 1 < NCHUNK)
      def _():
        pltpu.async_copy(
            flat_hbm.at[pl.ds(base + (cur + 1) * CHUNK, CHUNK)],
            bufs[1 - b],
            sems[1 - b],
        )

      pltpu.make_async_copy(
          flat_hbm.at[pl.ds(0, CHUNK)], bufs[b], sems[b]
      ).wait()
      compute(bufs[b], base + cur * CHUNK)

  pltpu.sync_copy(acc_v, psum_hbm.at[wid])


@functools.partial(
    pl.kernel,
    out_type=jax.ShapeDtypeStruct((NW, NSEG, D), jnp.float32),
    mesh=plsc.VectorSubcoreMesh(core_axis_name="c", subcore_axis_name="s"),
    scratch_types=[
        pltpu.VMEM((2 * LANES,), jnp.int32),
        pltpu.VMEM((CHUNK, D), jnp.float32),
        pltpu.VMEM((CHUNK, D), jnp.float32),
        pltpu.VMEM((NSEG, D), jnp.float32),
        pltpu.SemaphoreType.DMA,
        pltpu.SemaphoreType.DMA,
    ],
)
def _sc_segsum(flat_hbm, cu_hbm, psum_hbm, cu_v, buf0, buf1, acc_v, s0, s1):
  _sc_segsum_body(flat_hbm, cu_hbm, psum_hbm, cu_v, buf0, buf1, acc_v, s0, s1)


def _tc_segsum_body(cu_ref, flat_ref, out_ref):
  i = pl.program_id(0)
  rows = lax.broadcasted_iota(jnp.int32, (1, TBLK), 1) + i * TBLK
  segid = jnp.zeros((1, TBLK), jnp.int32)
  for s in range(1, NSEG):
    segid += (rows >= cu_ref[s]).astype(jnp.int32)
  onehot = (
      lax.broadcasted_iota(jnp.int32, (NSEG, TBLK), 0) == segid
  ).astype(jnp.float32)
  part = jnp.dot(
      onehot,
      flat_ref[...],
      preferred_element_type=jnp.float32,
      precision=lax.Precision.HIGHEST,
  )

  @pl.when(i == 0)
  def _():
    out_ref[...] = part

  @pl.when(i > 0)
  def _():
    out_ref[...] += part


_tc_segsum = pl.pallas_call(
    _tc_segsum_body,
    grid=(TC_STEPS,),
    out_shape=jax.ShapeDtypeStruct((NSEG, D), jnp.float32),
    in_specs=[
        pl.BlockSpec(memory_space=pltpu.SMEM),
        pl.BlockSpec((TBLK, D), lambda i: (i, 0)),
    ],
    out_specs=pl.BlockSpec((NSEG, D), lambda i: (0, 0)),
)


def _mlp_body(
    cu_ref, psum_ref, psum_tc_ref, w1_ref, b1_ref, w2_ref, b2_ref, out_ref
):
  sums = jnp.sum(psum_ref[...], axis=0) + psum_tc_ref[...]
  scales = []
  for s in range(NSEG):
    cnt = (cu_ref[s + 1] - cu_ref[s]).astype(jnp.float32)
    scales.append(jnp.full((1, D), 1.0 / jnp.maximum(cnt, 1.0), jnp.float32))
  mean = sums * jnp.concatenate(scales, axis=0)
  h = jnp.maximum(
      jnp.dot(
          mean,
          w1_ref[...],
          preferred_element_type=jnp.float32,
          precision=lax.Precision.HIGHEST,
      )
      + b1_ref[...],
      0.0,
  )
  out_ref[...] = (
      jnp.dot(
          h,
          w2_ref[...],
          preferred_element_type=jnp.float32,
          precision=lax.Precision.HIGHEST,
      )
      + b2_ref[...]
  )


_mlp_call = pl.pallas_call(
    _mlp_body,
    out_shape=jax.ShapeDtypeStruct((NSEG, D), jnp.float32),
    in_specs=[
        pl.BlockSpec(memory_space=pltpu.SMEM),
        pl.BlockSpec(memory_space=pltpu.VMEM),
        pl.BlockSpec(memory_space=pltpu.VMEM),
        pl.BlockSpec(memory_space=pltpu.VMEM),
        pl.BlockSpec(memory_space=pltpu.VMEM),
        pl.BlockSpec(memory_space=pltpu.VMEM),
        pl.BlockSpec(memory_space=pltpu.VMEM),
    ],
    out_specs=pl.BlockSpec(memory_space=pltpu.VMEM),
)


@jax.jit
def kernel(flat, cu_seqlens, W1, b1, W2, b2):
  psum_sc = _sc_segsum(flat, cu_seqlens)
  psum_tc = _tc_segsum(cu_seqlens, flat)
  return _mlp_call(
      cu_seqlens, psum_sc, psum_tc, W1, b1.reshape(1, -1), W2,
      b2.reshape(1, -1)
  )


# speedup vs baseline: 1.0134x; 1.0134x over previous
import functools

import jax
import jax.numpy as jnp
from jax import lax
from jax.experimental import pallas as pl
from jax.experimental.pallas import tpu as pltpu
from jax.experimental.pallas import tpu_sc as plsc

TOTAL = 32768
D = 256
NSEG = 16
LANES = 16
NVEC = D // LANES

TC_ROWS = 16384
SC_ROWS = TOTAL - TC_ROWS

NC = 2
NS = 16
NW = NC * NS
ROWS_PER_W = SC_ROWS // NW
CHUNK = 128
NCHUNK = ROWS_PER_W // CHUNK

TBLK = 2048
TC_STEPS = TC_ROWS // TBLK


def _sc_segsum_body(
    flat_hbm, cu_hbm, psum_hbm, cu_v, buf0, buf1, acc_v, sem0, sem1
):
  cid = lax.axis_index("c")
  sid = lax.axis_index("s")
  wid = sid * NC + cid
  base = TC_ROWS + wid * ROWS_PER_W

  pltpu.sync_copy(cu_hbm.at[pl.ds(0, LANES)], cu_v.at[pl.ds(0, LANES)])
  cu_v[pl.ds(LANES, LANES)] = jnp.full((LANES,), TOTAL, jnp.int32)

  def cu_at(k):
    return cu_v[pl.ds(k, LANES)][0]

  zeros = jnp.zeros((LANES,), jnp.float32)

  @pl.loop(0, NSEG)
  def _zero(s):
    for j in range(NVEC):
      acc_v[s, pl.ds(j * LANES, LANES)] = zeros

  bufs = (buf0, buf1)
  sems = (sem0, sem1)

  pltpu.async_copy(flat_hbm.at[pl.ds(base, CHUNK)], bufs[0], sems[0])

  def compute(buf, row0):
    @pl.loop(0, NSEG)
    def _seg(k):
      lo = jnp.maximum(cu_at(k), row0) - row0
      hi = jnp.minimum(cu_at(k + 1), row0 + CHUNK) - row0

      @pl.when(hi > lo)
      def _():
        init = tuple(acc_v[k, pl.ds(j * LANES, LANES)] for j in range(NVEC))

        @pl.loop(lo, hi, init_carry=init)
        def acc(r, carry):
          return tuple(
              carry[j] + buf[r, pl.ds(j * LANES, LANES)]
              for j in range(NVEC)
          )

        for j in range(NVEC):
          acc_v[k, pl.ds(j * LANES, LANES)] = acc[j]

  @pl.loop(0, NCHUNK, step=2)
  def _pair(ch):
    for b in range(2):
      cur = ch + b

      @pl.when(cur + 1 < NCHUNK)
      def _():
        pltpu.async_copy(
            flat_hbm.at[pl.ds(base + (cur + 1) * CHUNK, CHUNK)],
            bufs[1 - b],
            sems[1 - b],
        )

      pltpu.make_async_copy(
          flat_hbm.at[pl.ds(0, CHUNK)], bufs[b], sems[b]
      ).wait()
      compute(bufs[b], base + cur * CHUNK)

  pltpu.sync_copy(acc_v, psum_hbm.at[wid])


@functools.partial(
    pl.kernel,
    out_type=jax.ShapeDtypeStruct((NW, NSEG, D), jnp.float32),
    mesh=plsc.VectorSubcoreMesh(core_axis_name="c", subcore_axis_name="s"),
    scratch_types=[
        pltpu.VMEM((2 * LANES,), jnp.int32),
        pltpu.VMEM((CHUNK, D), jnp.float32),
        pltpu.VMEM((CHUNK, D), jnp.float32),
        pltpu.VMEM((NSEG, D), jnp.float32),
        pltpu.SemaphoreType.DMA,
        pltpu.SemaphoreType.DMA,
    ],
)
def _sc_segsum(flat_hbm, cu_hbm, psum_hbm, cu_v, buf0, buf1, acc_v, s0, s1):
  _sc_segsum_body(flat_hbm, cu_hbm, psum_hbm, cu_v, buf0, buf1, acc_v, s0, s1)


def _tc_segsum_body(cu_ref, flat_ref, out_ref):
  i = pl.program_id(0)
  rows = lax.broadcasted_iota(jnp.int32, (1, TBLK), 1) + i * TBLK
  segid = jnp.zeros((1, TBLK), jnp.int32)
  for s in range(1, NSEG):
    segid += (rows >= cu_ref[s]).astype(jnp.int32)
  onehot = (
      lax.broadcasted_iota(jnp.int32, (NSEG, TBLK), 0) == segid
  ).astype(jnp.float32)
  part = jnp.dot(
      onehot,
      flat_ref[...],
      preferred_element_type=jnp.float32,
      precision=lax.Precision.HIGHEST,
  )

  @pl.when(i == 0)
  def _():
    out_ref[...] = part

  @pl.when(i > 0)
  def _():
    out_ref[...] += part


_tc_segsum = pl.pallas_call(
    _tc_segsum_body,
    grid=(TC_STEPS,),
    out_shape=jax.ShapeDtypeStruct((NSEG, D), jnp.float32),
    in_specs=[
        pl.BlockSpec(memory_space=pltpu.SMEM),
        pl.BlockSpec((TBLK, D), lambda i: (i, 0)),
    ],
    out_specs=pl.BlockSpec((NSEG, D), lambda i: (0, 0)),
)


def _mlp_body(
    cu_ref, psum_ref, psum_tc_ref, w1_ref, b1_ref, w2_ref, b2_ref, out_ref
):
  sums = jnp.sum(psum_ref[...], axis=0) + psum_tc_ref[...]
  scales = []
  for s in range(NSEG):
    cnt = (cu_ref[s + 1] - cu_ref[s]).astype(jnp.float32)
    scales.append(jnp.full((1, D), 1.0 / jnp.maximum(cnt, 1.0), jnp.float32))
  mean = sums * jnp.concatenate(scales, axis=0)
  h = jnp.maximum(
      jnp.dot(
          mean,
          w1_ref[...],
          preferred_element_type=jnp.float32,
          precision=lax.Precision.HIGHEST,
      )
      + b1_ref[...],
      0.0,
  )
  out_ref[...] = (
      jnp.dot(
          h,
          w2_ref[...],
          preferred_element_type=jnp.float32,
          precision=lax.Precision.HIGHEST,
      )
      + b2_ref[...]
  )


_mlp_call = pl.pallas_call(
    _mlp_body,
    out_shape=jax.ShapeDtypeStruct((NSEG, D), jnp.float32),
    in_specs=[
        pl.BlockSpec(memory_space=pltpu.SMEM),
        pl.BlockSpec(memory_space=pltpu.VMEM),
        pl.BlockSpec(memory_space=pltpu.VMEM),
        pl.BlockSpec(memory_space=pltpu.VMEM),
        pl.BlockSpec(memory_space=pltpu.VMEM),
        pl.BlockSpec(memory_space=pltpu.VMEM),
        pl.BlockSpec(memory_space=pltpu.VMEM),
    ],
    out_specs=pl.BlockSpec(memory_space=pltpu.VMEM),
)


@jax.jit
def kernel(flat, cu_seqlens, W1, b1, W2, b2):
  psum_sc = _sc_segsum(flat, cu_seqlens)
  psum_tc = _tc_segsum(cu_seqlens, flat)
  return _mlp_call(
      cu_seqlens, psum_sc, psum_tc, W1, b1.reshape(1, -1), W2,
      b2.reshape(1, -1)
  )


# speedup vs baseline: 1.0395x; 1.0257x over previous
import functools

import jax
import jax.numpy as jnp
from jax import lax
from jax.experimental import pallas as pl
from jax.experimental.pallas import tpu as pltpu
from jax.experimental.pallas import tpu_sc as plsc

TOTAL = 32768
D = 256
NSEG = 16
LANES = 16
NVEC = D // LANES

TC_ROWS = 18432
SC_ROWS = TOTAL - TC_ROWS

NC = 2
NS = 16
NW = NC * NS
ROWS_PER_W = SC_ROWS // NW
CHUNK = 64
NCHUNK = ROWS_PER_W // CHUNK

TBLK = 2048
TC_STEPS = TC_ROWS // TBLK


def _sc_segsum_body(
    flat_hbm, cu_hbm, psum_hbm, cu_v, buf0, buf1, acc_v, sem0, sem1
):
  cid = lax.axis_index("c")
  sid = lax.axis_index("s")
  wid = sid * NC + cid
  base = TC_ROWS + wid * ROWS_PER_W

  pltpu.sync_copy(cu_hbm.at[pl.ds(0, LANES)], cu_v.at[pl.ds(0, LANES)])
  cu_v[pl.ds(LANES, LANES)] = jnp.full((LANES,), TOTAL, jnp.int32)

  def cu_at(k):
    return cu_v[pl.ds(k, LANES)][0]

  zeros = jnp.zeros((LANES,), jnp.float32)

  @pl.loop(0, NSEG)
  def _zero(s):
    for j in range(NVEC):
      acc_v[s, pl.ds(j * LANES, LANES)] = zeros

  bufs = (buf0, buf1)
  sems = (sem0, sem1)

  pltpu.async_copy(flat_hbm.at[pl.ds(base, CHUNK)], bufs[0], sems[0])

  def compute(buf, row0):
    @pl.loop(0, NSEG)
    def _seg(k):
      lo = jnp.maximum(cu_at(k), row0) - row0
      hi = jnp.minimum(cu_at(k + 1), row0 + CHUNK) - row0

      @pl.when(hi > lo)
      def _():
        init = tuple(acc_v[k, pl.ds(j * LANES, LANES)] for j in range(NVEC))

        @pl.loop(lo, hi, init_carry=init)
        def acc(r, carry):
          return tuple(
              carry[j] + buf[r, pl.ds(j * LANES, LANES)]
              for j in range(NVEC)
          )

        for j in range(NVEC):
          acc_v[k, pl.ds(j * LANES, LANES)] = acc[j]

  @pl.loop(0, NCHUNK, step=2)
  def _pair(ch):
    for b in range(2):
      cur = ch + b

      @pl.when(cur < NCHUNK)
      def _():
        @pl.when(cur + 1 < NCHUNK)
        def _():
          pltpu.async_copy(
              flat_hbm.at[pl.ds(base + (cur + 1) * CHUNK, CHUNK)],
              bufs[1 - b],
              sems[1 - b],
          )

        pltpu.make_async_copy(
            flat_hbm.at[pl.ds(0, CHUNK)], bufs[b], sems[b]
        ).wait()
        compute(bufs[b], base + cur * CHUNK)

  pltpu.sync_copy(acc_v, psum_hbm.at[wid])


@functools.partial(
    pl.kernel,
    out_type=jax.ShapeDtypeStruct((NW, NSEG, D), jnp.float32),
    mesh=plsc.VectorSubcoreMesh(core_axis_name="c", subcore_axis_name="s"),
    scratch_types=[
        pltpu.VMEM((2 * LANES,), jnp.int32),
        pltpu.VMEM((CHUNK, D), jnp.float32),
        pltpu.VMEM((CHUNK, D), jnp.float32),
        pltpu.VMEM((NSEG, D), jnp.float32),
        pltpu.SemaphoreType.DMA,
        pltpu.SemaphoreType.DMA,
    ],
)
def _sc_segsum(flat_hbm, cu_hbm, psum_hbm, cu_v, buf0, buf1, acc_v, s0, s1):
  _sc_segsum_body(flat_hbm, cu_hbm, psum_hbm, cu_v, buf0, buf1, acc_v, s0, s1)


def _tc_segsum_body(cu_ref, flat_ref, out_ref):
  i = pl.program_id(0)
  rows = lax.broadcasted_iota(jnp.int32, (1, TBLK), 1) + i * TBLK
  segid = jnp.zeros((1, TBLK), jnp.int32)
  for s in range(1, NSEG):
    segid += (rows >= cu_ref[s]).astype(jnp.int32)
  onehot = (
      lax.broadcasted_iota(jnp.int32, (NSEG, TBLK), 0) == segid
  ).astype(jnp.float32)
  part = jnp.dot(
      onehot,
      flat_ref[...],
      preferred_element_type=jnp.float32,
      precision=lax.Precision.HIGHEST,
  )

  @pl.when(i == 0)
  def _():
    out_ref[...] = part

  @pl.when(i > 0)
  def _():
    out_ref[...] += part


_tc_segsum = pl.pallas_call(
    _tc_segsum_body,
    grid=(TC_STEPS,),
    out_shape=jax.ShapeDtypeStruct((NSEG, D), jnp.float32),
    in_specs=[
        pl.BlockSpec(memory_space=pltpu.SMEM),
        pl.BlockSpec((TBLK, D), lambda i: (i, 0)),
    ],
    out_specs=pl.BlockSpec((NSEG, D), lambda i: (0, 0)),
)


def _mlp_body(
    cu_ref, psum_ref, psum_tc_ref, w1_ref, b1_ref, w2_ref, b2_ref, out_ref
):
  sums = jnp.sum(psum_ref[...], axis=0) + psum_tc_ref[...]
  scales = []
  for s in range(NSEG):
    cnt = (cu_ref[s + 1] - cu_ref[s]).astype(jnp.float32)
    scales.append(jnp.full((1, D), 1.0 / jnp.maximum(cnt, 1.0), jnp.float32))
  mean = sums * jnp.concatenate(scales, axis=0)
  h = jnp.maximum(
      jnp.dot(
          mean,
          w1_ref[...],
          preferred_element_type=jnp.float32,
          precision=lax.Precision.HIGHEST,
      )
      + b1_ref[...],
      0.0,
  )
  out_ref[...] = (
      jnp.dot(
          h,
          w2_ref[...],
          preferred_element_type=jnp.float32,
          precision=lax.Precision.HIGHEST,
      )
      + b2_ref[...]
  )


_mlp_call = pl.pallas_call(
    _mlp_body,
    out_shape=jax.ShapeDtypeStruct((NSEG, D), jnp.float32),
    in_specs=[
        pl.BlockSpec(memory_space=pltpu.SMEM),
        pl.BlockSpec(memory_space=pltpu.VMEM),
        pl.BlockSpec(memory_space=pltpu.VMEM),
        pl.BlockSpec(memory_space=pltpu.VMEM),
        pl.BlockSpec(memory_space=pltpu.VMEM),
        pl.BlockSpec(memory_space=pltpu.VMEM),
        pl.BlockSpec(memory_space=pltpu.VMEM),
    ],
    out_specs=pl.BlockSpec(memory_space=pltpu.VMEM),
)


@jax.jit
def kernel(flat, cu_seqlens, W1, b1, W2, b2):
  psum_sc = _sc_segsum(flat, cu_seqlens)
  psum_tc = _tc_segsum(cu_seqlens, flat)
  return _mlp_call(
      cu_seqlens, psum_sc, psum_tc, W1, b1.reshape(1, -1), W2,
      b2.reshape(1, -1)
  )
